# Initial kernel scaffold; baseline (speedup 1.0000x reference)
import jax
import jax.numpy as jnp
from jax.experimental import pallas as pl


def kernel(filtre, X, node_indicator, W, b, Wc, bc):
    raise NotImplementedError("write your pallas kernel here")


# speedup vs baseline: 1.5449x; 1.5449x over previous
import jax
import jax.numpy as jnp
from jax.experimental import pallas as pl
from jax.experimental.pallas import tpu as pltpu

_N = 10000
_D = 128
_F = 4
_G = 128
_C = 10
_BM = 400
_NBLK = _N // _BM


def _fused(filtre_ref, x_ref, ind_ref, w_ref, b_ref, wc_ref, bc_ref,
           out_ref, y_scr, pool_scr):
    i = pl.program_id(0)

    @pl.when(i == 0)
    def _init():
        y_scr[...] = jnp.dot(x_ref[...], w_ref[...],
                             preferred_element_type=jnp.float32)
        pool_scr[...] = jnp.zeros_like(pool_scr)

    fblk = filtre_ref[...]
    h = jnp.dot(fblk, y_scr[...],
                preferred_element_type=jnp.float32)
    h = jnp.maximum(h + b_ref[...], 0.0)

    ind = ind_ref[...]
    gids = jax.lax.broadcasted_iota(jnp.int32, (_BM, _G), 1)
    mask = ind == gids
    for f in range(_F):
        vals = jnp.where(mask, h[:, f:f + 1], 0.0)
        part = jnp.max(vals, axis=0, keepdims=True)
        pool_scr[f:f + 1, :] = jnp.maximum(pool_scr[f:f + 1, :], part)

    @pl.when(i == _NBLK - 1)
    def _fin():
        logits = jax.lax.dot_general(
            pool_scr[...], wc_ref[...], (((0,), (0,)), ((), ())),
            preferred_element_type=jnp.float32) + bc_ref[...]
        m = jnp.max(logits, axis=1, keepdims=True)
        e = jnp.exp(logits - m)
        out_ref[...] = e / jnp.sum(e, axis=1, keepdims=True)


def kernel(filtre, X, node_indicator, W, b, Wc, bc):
    ind2d = node_indicator.astype(jnp.int32).reshape(_N, 1)
    b2d = b.reshape(1, _F)
    bc2d = bc.reshape(1, _C)
    return pl.pallas_call(
        _fused,
        grid=(_NBLK,),
        in_specs=[
            pl.BlockSpec((_BM, _N), lambda i: (i, 0)),
            pl.BlockSpec((_N, _D), lambda i: (0, 0)),
            pl.BlockSpec((_BM, 1), lambda i: (i, 0)),
            pl.BlockSpec((_D, _F), lambda i: (0, 0)),
            pl.BlockSpec((1, _F), lambda i: (0, 0)),
            pl.BlockSpec((_F, _C), lambda i: (0, 0)),
            pl.BlockSpec((1, _C), lambda i: (0, 0)),
        ],
        out_specs=pl.BlockSpec((_G, _C), lambda i: (0, 0)),
        out_shape=jax.ShapeDtypeStruct((_G, _C), jnp.float32),
        scratch_shapes=[
            pltpu.VMEM((_N, _F), jnp.float32),
            pltpu.VMEM((_F, _G), jnp.float32),
        ],
        compiler_params=pltpu.CompilerParams(
            dimension_semantics=("arbitrary",),
            vmem_limit_bytes=100 * 1024 * 1024,
        ),
    )(filtre, X, ind2d, W, b2d, Wc, bc2d)
